# baseline (speedup 1.0000x reference)
import functools

import jax
import jax.numpy as jnp
from jax import lax
from jax.experimental import pallas as pl
from jax.experimental.pallas import tpu as pltpu
from jax.experimental.pallas import tpu_sc as plsc

_B = 16384
_D = 32
_NC = 2
_NS = 16
_NW = _NC * _NS
_BPW = _B // _NW


def _sc_body(uidx_hbm, iidx_hbm, user_hbm, item_hbm, out_hbm,
             uidx_v, iidx_v, urows_v, irows_v, out_v, sem_u, sem_i):
    wid = lax.axis_index("s") * _NC + lax.axis_index("c")
    base = wid * _BPW

    pltpu.sync_copy(uidx_hbm.at[pl.ds(base, _BPW)], uidx_v)
    pltpu.sync_copy(iidx_hbm.at[pl.ds(base, _BPW)], iidx_v)
    cu = pltpu.async_copy(user_hbm.at[uidx_v], urows_v, sem_u)
    ci = pltpu.async_copy(item_hbm.at[iidx_v], irows_v, sem_i)
    cu.wait()
    ci.wait()

    lanes = lax.iota(jnp.int32, 16)

    def group(g, _):
        rows = jnp.full((16,), g * 16, jnp.int32) + lanes
        acc = jnp.zeros((16,), jnp.float32)
        for d in range(_D):
            cols = (lanes + d) & (_D - 1)
            gu = plsc.load_gather(urows_v, [rows, cols])
            gi = plsc.load_gather(irows_v, [rows, cols])
            acc = acc + gu * gi
        out_v[pl.ds(g * 16, 16)] = acc
        return _

    lax.fori_loop(0, _BPW // 16, group, None)
    pltpu.sync_copy(out_v, out_hbm.at[pl.ds(base, _BPW)])


@functools.partial(jax.jit, static_argnames=())
def _sc_call(uidx, iidx, user_table, item_table):
    mesh = plsc.VectorSubcoreMesh(core_axis_name="c", subcore_axis_name="s")
    fn = functools.partial(
        pl.kernel,
        mesh=mesh,
        out_type=jax.ShapeDtypeStruct((_B,), jnp.float32),
        scratch_types=[
            pltpu.VMEM((_BPW,), jnp.int32),
            pltpu.VMEM((_BPW,), jnp.int32),
            pltpu.VMEM((_BPW, _D), jnp.float32),
            pltpu.VMEM((_BPW, _D), jnp.float32),
            pltpu.VMEM((_BPW,), jnp.float32),
            pltpu.SemaphoreType.DMA,
            pltpu.SemaphoreType.DMA,
        ],
        compiler_params=pltpu.CompilerParams(
            needs_layout_passes=False, use_tc_tiling_on_sc=False),
    )(_sc_body)
    return fn(uidx, iidx, user_table, item_table)


def kernel(input_tensor, user_table, item_table):
    uidx = input_tensor[:, 0].astype(jnp.int32)
    iidx = input_tensor[:, 1].astype(jnp.int32)
    out = _sc_call(uidx, iidx, user_table, item_table)
    return out[:, None]


# speedup vs baseline: 3.6054x; 3.6054x over previous
import functools

import jax
import jax.numpy as jnp
from jax import lax
from jax.experimental import pallas as pl
from jax.experimental.pallas import tpu as pltpu
from jax.experimental.pallas import tpu_sc as plsc

_B = 16384
_D = 32
_NC = 2
_NS = 16
_NW = _NC * _NS
_BPW = _B // _NW
_CH = 4
_NCHUNK = _BPW // _CH


def _sc_body(uidx_hbm, iidx_hbm, ut_hbm, it_hbm, out_hbm,
             uidx_v, iidx_v, ubuf, ibuf, out_v,
             sem0, sem1):
    wid = lax.axis_index("s") * _NC + lax.axis_index("c")

    pltpu.sync_copy(uidx_hbm.at[wid], uidx_v)
    pltpu.sync_copy(iidx_hbm.at[wid], iidx_v)

    lanes0 = lax.iota(jnp.int32, 16)

    def scalar_idx(idx_v, item):
        vec = idx_v[pl.ds((item // 16) * 16, 16)]
        return jnp.sum(jnp.where(lanes0 == (item & 15), vec, 0))

    def fire(c, pb, sem):
        for j in range(_CH):
            item = c * _CH + j
            ur = scalar_idx(uidx_v, item)
            cu = pl.multiple_of((ur >> 7) * 128, 128)
            pltpu.async_copy(ut_hbm.at[:, pl.ds(cu, 128)],
                             ubuf.at[pb, j], sem)
            ir = scalar_idx(iidx_v, item)
            ci = pl.multiple_of((ir >> 7) * 128, 128)
            pltpu.async_copy(it_hbm.at[:, pl.ds(ci, 128)],
                             ibuf.at[pb, j], sem)

    def drain(pb, sem):
        for j in range(_CH):
            pltpu.make_async_copy(ut_hbm.at[:, pl.ds(0, 128)],
                                  ubuf.at[pb, j], sem).wait()
            pltpu.make_async_copy(it_hbm.at[:, pl.ds(0, 128)],
                                  ibuf.at[pb, j], sem).wait()

    lanes = lax.iota(jnp.int32, 16)

    def compute(c, pb, acc):
        pbv = jnp.full((16,), pb, jnp.int32)
        for j in range(_CH):
            item = c * _CH + j
            jv = jnp.full((16,), j, jnp.int32)
            cu = jnp.full((16,), scalar_idx(uidx_v, item) & 127, jnp.int32)
            ci = jnp.full((16,), scalar_idx(iidx_v, item) & 127, jnp.int32)
            gu1 = plsc.load_gather(ubuf, [pbv, jv, lanes, cu])
            gu2 = plsc.load_gather(ubuf, [pbv, jv, lanes + 16, cu])
            gi1 = plsc.load_gather(ibuf, [pbv, jv, lanes, ci])
            gi2 = plsc.load_gather(ibuf, [pbv, jv, lanes + 16, ci])
            p = gu1 * gi1 + gu2 * gi2
            s = jnp.sum(p)
            acc = jnp.where(lanes == (item & 15), jnp.full((16,), s), acc)
        return acc

    fire(0, 0, sem0)

    def step(k, acc):
        c0 = 2 * k
        fire(c0 + 1, 1, sem1)
        drain(0, sem0)
        acc = compute(c0, 0, acc)

        @pl.when(c0 + 2 < _NCHUNK)
        def _():
            fire(c0 + 2, 0, sem0)

        drain(1, sem1)
        acc = compute(c0 + 1, 1, acc)

        @pl.when(k % 2 == 1)
        def _():
            out_v[pl.ds((k // 2) * 16, 16)] = acc

        return acc

    lax.fori_loop(0, _NCHUNK // 2, step, jnp.zeros((16,), jnp.float32))
    pltpu.sync_copy(out_v, out_hbm.at[wid])


@jax.jit
def _sc_call(uidx, iidx, ut, it):
    mesh = plsc.VectorSubcoreMesh(core_axis_name="c", subcore_axis_name="s")
    fn = functools.partial(
        pl.kernel,
        mesh=mesh,
        out_type=jax.ShapeDtypeStruct((_NW, _BPW), jnp.float32),
        scratch_types=[
            pltpu.VMEM((_BPW,), jnp.int32),
            pltpu.VMEM((_BPW,), jnp.int32),
            pltpu.VMEM((2, _CH, _D, 128), jnp.float32),
            pltpu.VMEM((2, _CH, _D, 128), jnp.float32),
            pltpu.VMEM((_BPW,), jnp.float32),
            pltpu.SemaphoreType.DMA,
            pltpu.SemaphoreType.DMA,
        ],
        compiler_params=pltpu.CompilerParams(needs_layout_passes=False),
    )(_sc_body)
    return fn(uidx, iidx, ut, it)


def kernel(input_tensor, user_table, item_table):
    uidx = input_tensor[:, 0].astype(jnp.int32).reshape(_NW, _BPW)
    iidx = input_tensor[:, 1].astype(jnp.int32).reshape(_NW, _BPW)
    out = _sc_call(uidx, iidx, user_table.T, item_table.T)
    return out.reshape(_B, 1)
